# baseline (speedup 1.0000x reference)
import functools

import jax
import jax.numpy as jnp
from jax import lax
from jax.experimental import pallas as pl
from jax.experimental.pallas import tpu as pltpu
from jax.experimental.pallas import tpu_sc as plsc

T_BLK = 256
I_BLK1 = 1024
I_BLK2 = 1024


@functools.cache
def _make_sc_row_scatter(n_in: int, n_out: int, d: int):
    info = plsc.get_sparse_core_info()
    nc, ns = info.num_cores, info.num_subcores
    nw = nc * ns
    assert n_in % (8 * nw) == 0
    rows_per_w = n_in // nw
    mesh = plsc.VectorSubcoreMesh(core_axis_name="c", subcore_axis_name="s")

    half = rows_per_w // 2

    @functools.partial(
        pl.kernel,
        mesh=mesh,
        out_type=jax.ShapeDtypeStruct((n_out, d), jnp.float32),
        scratch_types=[
            pltpu.VMEM((half,), jnp.int32),
            pltpu.VMEM((half,), jnp.int32),
            pltpu.VMEM((half, d), jnp.float32),
            pltpu.VMEM((half, d), jnp.float32),
            pltpu.SemaphoreType.DMA,
            pltpu.SemaphoreType.DMA,
            pltpu.SemaphoreType.DMA,
            pltpu.SemaphoreType.DMA,
        ],
    )
    def scatter_k(x_hbm, idx_hbm, out_hbm, idx0, idx1, buf0, buf1,
                  si0, si1, so0, so1):
        wid = lax.axis_index("s") * nc + lax.axis_index("c")
        base = wid * rows_per_w
        pltpu.sync_copy(idx_hbm.at[pl.ds(base, half)], idx0)
        pltpu.sync_copy(idx_hbm.at[pl.ds(base + half, half)], idx1)
        r0 = pltpu.async_copy(x_hbm.at[pl.ds(base, half)], buf0, si0)
        r1 = pltpu.async_copy(x_hbm.at[pl.ds(base + half, half)], buf1, si1)
        r0.wait()
        w0 = pltpu.async_copy(buf0, out_hbm.at[idx0], so0)
        r1.wait()
        w1 = pltpu.async_copy(buf1, out_hbm.at[idx1], so1)
        w0.wait()
        w1.wait()

    return scatter_k


@functools.cache
def _make_sc_row_gather(n_out: int, d: int):
    info = plsc.get_sparse_core_info()
    nc, ns = info.num_cores, info.num_subcores
    nw = nc * ns
    assert n_out % (8 * nw) == 0
    rows_per_w = n_out // nw
    mesh = plsc.VectorSubcoreMesh(core_axis_name="c", subcore_axis_name="s")

    @functools.partial(
        pl.kernel,
        mesh=mesh,
        out_type=jax.ShapeDtypeStruct((n_out, d), jnp.float32),
        scratch_types=[
            pltpu.VMEM((rows_per_w,), jnp.int32),
            pltpu.VMEM((rows_per_w, d), jnp.float32),
            pltpu.SemaphoreType.DMA,
        ],
    )
    def gather_k(table_hbm, idx_hbm, out_hbm, idx_v, rows_v, sem):
        wid = lax.axis_index("s") * nc + lax.axis_index("c")
        base = wid * rows_per_w
        pltpu.sync_copy(idx_hbm.at[pl.ds(base, rows_per_w)], idx_v)
        pltpu.async_copy(table_hbm.at[idx_v], rows_v, sem).wait()
        pltpu.sync_copy(rows_v, out_hbm.at[pl.ds(base, rows_per_w)])

    return gather_k


def _gateup_body(gids_ref, x_ref, wgv_ref, wgl_ref, wuv_ref, wul_ref, h_ref,
                 wbf_ref):
    b = pl.program_id(1)
    is_lang = gids_ref[b] != 0

    @pl.when(b == 0)
    def _():
        wbf_ref[0] = wgv_ref[...].astype(jnp.bfloat16)
        wbf_ref[1] = wgl_ref[...].astype(jnp.bfloat16)
        wbf_ref[2] = wuv_ref[...].astype(jnp.bfloat16)
        wbf_ref[3] = wul_ref[...].astype(jnp.bfloat16)

    x = x_ref[...].astype(jnp.bfloat16)

    def _stage(e):
        g = jnp.dot(x, wbf_ref[e], preferred_element_type=jnp.float32)
        u = jnp.dot(x, wbf_ref[e + 2], preferred_element_type=jnp.float32)
        h_ref[...] = ((g * jax.nn.sigmoid(g)) * u).astype(jnp.bfloat16)

    @pl.when(is_lang)
    def _():
        _stage(1)

    @pl.when(jnp.logical_not(is_lang))
    def _():
        _stage(0)


def _gateup(gids, xs, wg_v, wg_l, wu_v, wu_l):
    c, h_dim = xs.shape
    i_dim = wg_v.shape[1]
    nb = c // T_BLK
    nj = i_dim // I_BLK1
    grid_spec = pltpu.PrefetchScalarGridSpec(
        num_scalar_prefetch=1,
        grid=(nj, nb),
        in_specs=[
            pl.BlockSpec((T_BLK, h_dim), lambda j, b, g: (b, 0)),
            pl.BlockSpec((h_dim, I_BLK1), lambda j, b, g: (0, j)),
            pl.BlockSpec((h_dim, I_BLK1), lambda j, b, g: (0, j)),
            pl.BlockSpec((h_dim, I_BLK1), lambda j, b, g: (0, j)),
            pl.BlockSpec((h_dim, I_BLK1), lambda j, b, g: (0, j)),
        ],
        out_specs=pl.BlockSpec((T_BLK, I_BLK1), lambda j, b, g: (b, j)),
        scratch_shapes=[pltpu.VMEM((4, h_dim, I_BLK1), jnp.bfloat16)],
    )
    return pl.pallas_call(
        _gateup_body,
        grid_spec=grid_spec,
        out_shape=jax.ShapeDtypeStruct((c, i_dim), jnp.bfloat16),
        compiler_params=pltpu.CompilerParams(
            dimension_semantics=("arbitrary", "arbitrary"),
        ),
    )(gids, xs, wg_v, wg_l, wu_v, wu_l)


def _down_body(gids_ref, h_ref, wdv_ref, wdl_ref, o_ref, wdbf_ref):
    b = pl.program_id(0)
    is_lang = gids_ref[b] != 0

    @pl.when(b == 0)
    def _():
        wdbf_ref[0] = wdv_ref[...].astype(jnp.bfloat16)
        wdbf_ref[1] = wdl_ref[...].astype(jnp.bfloat16)

    h = h_ref[...]

    def _go(e):
        o_ref[...] = jnp.dot(
            h, wdbf_ref[e], preferred_element_type=jnp.float32
        )

    @pl.when(is_lang)
    def _():
        _go(1)

    @pl.when(jnp.logical_not(is_lang))
    def _():
        _go(0)


def _down(gids, h, wd_v, wd_l):
    c, i_dim = h.shape
    h_dim = wd_v.shape[1]
    nb = c // T_BLK
    grid_spec = pltpu.PrefetchScalarGridSpec(
        num_scalar_prefetch=1,
        grid=(nb,),
        in_specs=[
            pl.BlockSpec((T_BLK, i_dim), lambda b, g: (b, 0)),
            pl.BlockSpec((i_dim, h_dim), lambda b, g: (0, 0)),
            pl.BlockSpec((i_dim, h_dim), lambda b, g: (0, 0)),
        ],
        out_specs=pl.BlockSpec((T_BLK, h_dim), lambda b, g: (b, 0)),
        scratch_shapes=[pltpu.VMEM((2, i_dim, h_dim), jnp.bfloat16)],
    )
    return pl.pallas_call(
        _down_body,
        grid_spec=grid_spec,
        out_shape=jax.ShapeDtypeStruct((c, h_dim), jnp.float32),
        compiler_params=pltpu.CompilerParams(
            dimension_semantics=("arbitrary",),
        ),
    )(gids, h, wd_v, wd_l)


def kernel(hidden_states, token_type_ids, Wg_v, Wu_v, Wd_v, Wg_l, Wu_l, Wd_l):
    b, s, h_dim = hidden_states.shape
    n = b * s
    x2d = hidden_states.reshape(n, h_dim)
    cap = n + T_BLK
    nb = cap // T_BLK

    tt = token_type_ids.astype(jnp.int32)
    left = tt[:, :-1] == 1
    right = tt[:, 1:] == 1
    vis = jnp.concatenate(
        [left & right, jnp.zeros((b, 1), dtype=bool)], axis=1
    ).reshape(n)
    nv = jnp.sum(vis.astype(jnp.int32))
    nv_pad = ((nv + T_BLK - 1) // T_BLK) * T_BLK
    vis_rank = jnp.cumsum(vis.astype(jnp.int32)) - 1
    lang_rank = jnp.cumsum((~vis).astype(jnp.int32)) - 1
    slot = jnp.where(vis, vis_rank, nv_pad + lang_rank).astype(jnp.int32)
    gids = (jnp.arange(nb, dtype=jnp.int32) * T_BLK >= nv_pad).astype(jnp.int32)

    xs = _make_sc_row_scatter(n, cap, h_dim)(x2d, slot)
    hs = _gateup(gids, xs, Wg_v, Wg_l, Wu_v, Wu_l)
    ys = _down(gids, hs, Wd_v, Wd_l)
    out2d = _make_sc_row_gather(n, h_dim)(ys, slot)
    return out2d.reshape(b, s, h_dim)


# speedup vs baseline: 1.0186x; 1.0186x over previous
import functools

import jax
import jax.numpy as jnp
from jax import lax
from jax.experimental import pallas as pl
from jax.experimental.pallas import tpu as pltpu
from jax.experimental.pallas import tpu_sc as plsc

T_BLK = 256
I_BLK1 = 1024
I_BLK2 = 1024


@functools.cache
def _make_sc_row_scatter(n_in: int, n_out: int, d: int):
    info = plsc.get_sparse_core_info()
    nc, ns = info.num_cores, info.num_subcores
    nw = nc * ns
    assert n_in % (8 * nw) == 0
    rows_per_w = n_in // nw
    mesh = plsc.VectorSubcoreMesh(core_axis_name="c", subcore_axis_name="s")

    half = rows_per_w // 2

    @functools.partial(
        pl.kernel,
        mesh=mesh,
        out_type=jax.ShapeDtypeStruct((n_out, d), jnp.float32),
        scratch_types=[
            pltpu.VMEM((half,), jnp.int32),
            pltpu.VMEM((half,), jnp.int32),
            pltpu.VMEM((half, d), jnp.float32),
            pltpu.VMEM((half, d), jnp.float32),
            pltpu.SemaphoreType.DMA,
            pltpu.SemaphoreType.DMA,
            pltpu.SemaphoreType.DMA,
            pltpu.SemaphoreType.DMA,
        ],
    )
    def scatter_k(x_hbm, idx_hbm, out_hbm, idx0, idx1, buf0, buf1,
                  si0, si1, so0, so1):
        wid = lax.axis_index("s") * nc + lax.axis_index("c")
        base = wid * rows_per_w
        pltpu.sync_copy(idx_hbm.at[pl.ds(base, half)], idx0)
        pltpu.sync_copy(idx_hbm.at[pl.ds(base + half, half)], idx1)
        r0 = pltpu.async_copy(x_hbm.at[pl.ds(base, half)], buf0, si0)
        r1 = pltpu.async_copy(x_hbm.at[pl.ds(base + half, half)], buf1, si1)
        r0.wait()
        w0 = pltpu.async_copy(buf0, out_hbm.at[idx0], so0)
        r1.wait()
        w1 = pltpu.async_copy(buf1, out_hbm.at[idx1], so1)
        w0.wait()
        w1.wait()

    return scatter_k


@functools.cache
def _make_sc_row_gather(n_out: int, d: int):
    info = plsc.get_sparse_core_info()
    nc, ns = info.num_cores, info.num_subcores
    nw = nc * ns
    assert n_out % (8 * nw) == 0
    rows_per_w = n_out // nw
    mesh = plsc.VectorSubcoreMesh(core_axis_name="c", subcore_axis_name="s")

    @functools.partial(
        pl.kernel,
        mesh=mesh,
        out_type=jax.ShapeDtypeStruct((n_out, d), jnp.float32),
        scratch_types=[
            pltpu.VMEM((rows_per_w,), jnp.int32),
            pltpu.VMEM((rows_per_w, d), jnp.float32),
            pltpu.SemaphoreType.DMA,
        ],
    )
    def gather_k(table_hbm, idx_hbm, out_hbm, idx_v, rows_v, sem):
        wid = lax.axis_index("s") * nc + lax.axis_index("c")
        base = wid * rows_per_w
        pltpu.sync_copy(idx_hbm.at[pl.ds(base, rows_per_w)], idx_v)
        pltpu.async_copy(table_hbm.at[idx_v], rows_v, sem).wait()
        pltpu.sync_copy(rows_v, out_hbm.at[pl.ds(base, rows_per_w)])

    return gather_k


def _gateup_body(gids_ref, x_ref, wgv_ref, wgl_ref, wuv_ref, wul_ref, h_ref):
    b = pl.program_id(1)
    x = x_ref[...]
    is_lang = gids_ref[b] != 0

    def _stage(wg_ref, wu_ref):
        g = jnp.dot(x, wg_ref[...], preferred_element_type=jnp.float32)
        u = jnp.dot(x, wu_ref[...], preferred_element_type=jnp.float32)
        h_ref[...] = ((g * jax.nn.sigmoid(g)) * u).astype(jnp.bfloat16)

    @pl.when(is_lang)
    def _():
        _stage(wgl_ref, wul_ref)

    @pl.when(jnp.logical_not(is_lang))
    def _():
        _stage(wgv_ref, wuv_ref)


def _gateup(gids, xs, wg_v, wg_l, wu_v, wu_l):
    c, h_dim = xs.shape
    i_dim = wg_v.shape[1]
    nb = c // T_BLK
    nj = i_dim // I_BLK1
    grid_spec = pltpu.PrefetchScalarGridSpec(
        num_scalar_prefetch=1,
        grid=(nj, nb),
        in_specs=[
            pl.BlockSpec((T_BLK, h_dim), lambda j, b, g: (b, 0)),
            pl.BlockSpec((h_dim, I_BLK1), lambda j, b, g: (0, j)),
            pl.BlockSpec((h_dim, I_BLK1), lambda j, b, g: (0, j)),
            pl.BlockSpec((h_dim, I_BLK1), lambda j, b, g: (0, j)),
            pl.BlockSpec((h_dim, I_BLK1), lambda j, b, g: (0, j)),
        ],
        out_specs=pl.BlockSpec((T_BLK, I_BLK1), lambda j, b, g: (b, j)),
    )
    return pl.pallas_call(
        _gateup_body,
        grid_spec=grid_spec,
        out_shape=jax.ShapeDtypeStruct((c, i_dim), jnp.bfloat16),
        compiler_params=pltpu.CompilerParams(
            dimension_semantics=("arbitrary", "arbitrary"),
        ),
    )(gids, xs, wg_v, wg_l, wu_v, wu_l)


def _down_body(gids_ref, h_ref, wdv_ref, wdl_ref, o_ref, wdbf_ref):
    b = pl.program_id(0)
    is_lang = gids_ref[b] != 0

    @pl.when(b == 0)
    def _():
        wdbf_ref[0] = wdv_ref[...].astype(jnp.bfloat16)
        wdbf_ref[1] = wdl_ref[...].astype(jnp.bfloat16)

    h = h_ref[...]

    def _go(e):
        o_ref[...] = jnp.dot(
            h, wdbf_ref[e], preferred_element_type=jnp.float32
        )

    @pl.when(is_lang)
    def _():
        _go(1)

    @pl.when(jnp.logical_not(is_lang))
    def _():
        _go(0)


def _down(gids, h, wd_v, wd_l):
    c, i_dim = h.shape
    h_dim = wd_v.shape[1]
    nb = c // T_BLK
    grid_spec = pltpu.PrefetchScalarGridSpec(
        num_scalar_prefetch=1,
        grid=(nb,),
        in_specs=[
            pl.BlockSpec((T_BLK, i_dim), lambda b, g: (b, 0)),
            pl.BlockSpec((i_dim, h_dim), lambda b, g: (0, 0)),
            pl.BlockSpec((i_dim, h_dim), lambda b, g: (0, 0)),
        ],
        out_specs=pl.BlockSpec((T_BLK, h_dim), lambda b, g: (b, 0)),
        scratch_shapes=[pltpu.VMEM((2, i_dim, h_dim), jnp.bfloat16)],
    )
    return pl.pallas_call(
        _down_body,
        grid_spec=grid_spec,
        out_shape=jax.ShapeDtypeStruct((c, h_dim), jnp.float32),
        compiler_params=pltpu.CompilerParams(
            dimension_semantics=("arbitrary",),
        ),
    )(gids, h, wd_v, wd_l)


def kernel(hidden_states, token_type_ids, Wg_v, Wu_v, Wd_v, Wg_l, Wu_l, Wd_l):
    b, s, h_dim = hidden_states.shape
    n = b * s
    x2d = hidden_states.reshape(n, h_dim)
    cap = n + T_BLK
    nb = cap // T_BLK

    tt = token_type_ids.astype(jnp.int32)
    left = tt[:, :-1] == 1
    right = tt[:, 1:] == 1
    vis = jnp.concatenate(
        [left & right, jnp.zeros((b, 1), dtype=bool)], axis=1
    ).reshape(n)
    nv = jnp.sum(vis.astype(jnp.int32))
    nv_pad = ((nv + T_BLK - 1) // T_BLK) * T_BLK
    vis_rank = jnp.cumsum(vis.astype(jnp.int32)) - 1
    lang_rank = jnp.cumsum((~vis).astype(jnp.int32)) - 1
    slot = jnp.where(vis, vis_rank, nv_pad + lang_rank).astype(jnp.int32)
    gids = (jnp.arange(nb, dtype=jnp.int32) * T_BLK >= nv_pad).astype(jnp.int32)

    xs = _make_sc_row_scatter(n, cap, h_dim)(x2d, slot)
    hs = _gateup(gids, xs, Wg_v, Wg_l, Wu_v, Wu_l)
    ys = _down(gids, hs, Wd_v, Wd_l)
    out2d = _make_sc_row_gather(n, h_dim)(ys, slot)
    return out2d.reshape(b, s, h_dim)
